# baseline (speedup 1.0000x reference)
import dataclasses
import functools

import jax
import jax.numpy as jnp
from jax import lax
from jax.experimental import pallas as pl
from jax.experimental.pallas import tpu as pltpu
from jax.experimental.pallas import tpu_sc as plsc

_B = 4096
_C = 1000
_G = 4
_R = _B // _G

_NC = 2
_NS = 16
_L = 16
_NW = _NC * _NS
_PW = _B // _NW
_CT = 1024


def _hist_sc_body(y_hbm, out_hbm, idx_v, table_v):
    cid = lax.axis_index("c")
    sid = lax.axis_index("s")
    wid = sid * _NC + cid
    base = wid * _PW
    pltpu.sync_copy(y_hbm.at[pl.ds(base, _PW)], idx_v)
    zero = jnp.zeros((_L,), jnp.float32)
    for j in range(_CT // _L):
        table_v[pl.ds(j * _L, _L)] = zero
    ones = jnp.ones((_L,), jnp.float32)
    for j in range(_PW // _L):
        iv = idx_v[pl.ds(j * _L, _L)]
        plsc.addupdate_scatter(table_v, [iv], ones)
    pltpu.sync_copy(table_v, out_hbm.at[wid])


def _hist_sc(y):
    mesh = plsc.VectorSubcoreMesh(core_axis_name="c", subcore_axis_name="s")
    cp = pltpu.CompilerParams()
    if "needs_layout_passes" in pltpu.CompilerParams.__dataclass_fields__:
        cp = dataclasses.replace(cp, needs_layout_passes=False)
    k = pl.kernel(
        _hist_sc_body,
        out_type=jax.ShapeDtypeStruct((_NW, _CT), jnp.float32),
        mesh=mesh,
        scratch_types=[
            pltpu.VMEM((_PW,), jnp.int32),
            pltpu.VMEM((_CT,), jnp.float32),
        ],
        compiler_params=cp,
    )
    return k(y)


def _rce_tc_kernel(x_ref, hist_ref, out_ref, colsum_acc):
    i = pl.program_id(0)

    @pl.when(i == 0)
    def _init():
        colsum_acc[...] = jnp.zeros_like(colsum_acc)

    colsum_acc[...] += jnp.sum(x_ref[...], axis=0, keepdims=True)

    @pl.when(i == _G - 1)
    def _final():
        counts = jnp.sum(hist_ref[...], axis=0, keepdims=True)
        s = jnp.sum(colsum_acc[...] * counts[:, :_C], keepdims=True)
        out_ref[...] = 6.0 - (6.0 / (_B * _B)) * s


def kernel(x, y):
    hist = _hist_sc(y.astype(jnp.int32))
    out = pl.pallas_call(
        _rce_tc_kernel,
        grid=(_G,),
        in_specs=[
            pl.BlockSpec((_R, _C), lambda i: (i, 0)),
            pl.BlockSpec((_NW, _CT), lambda i: (0, 0)),
        ],
        out_specs=pl.BlockSpec((1, 1), lambda i: (0, 0)),
        out_shape=jax.ShapeDtypeStruct((1, 1), jnp.float32),
        scratch_shapes=[
            pltpu.VMEM((1, _C), jnp.float32),
        ],
    )(x, hist)
    return jnp.reshape(out, ())
